# baseline (speedup 1.0000x reference)
import functools

import jax
import jax.numpy as jnp
from jax import lax
from jax.experimental import pallas as pl
from jax.experimental.pallas import tpu as pltpu
from jax.experimental.pallas import tpu_sc as plsc


def kernel(x):
    B, T, F = x.shape
    rows = B * T
    x2 = x.reshape(rows, F)

    info = plsc.get_sparse_core_info()
    NC, NS = info.num_cores, info.num_subcores
    NW = NC * NS
    rpw = rows // NW

    mesh = plsc.VectorSubcoreMesh(core_axis_name="c", subcore_axis_name="s")

    @functools.partial(
        pl.kernel,
        mesh=mesh,
        out_type=jax.ShapeDtypeStruct((rows, F), x.dtype),
    )
    def sc_copy(x_hbm, out_hbm):
        wid = lax.axis_index("s") * NC + lax.axis_index("c")
        base = wid * rpw
        pltpu.sync_copy(
            x_hbm.at[pl.ds(base, rpw), :],
            out_hbm.at[pl.ds(base, rpw), :],
        )

    return sc_copy(x2).reshape(B, T, F)


# speedup vs baseline: 49.2064x; 49.2064x over previous
import jax
import jax.numpy as jnp
from jax.experimental import pallas as pl


def _copy_kernel(x_ref, o_ref):
    o_ref[...] = x_ref[...]


def kernel(x):
    B, T, F = x.shape
    rows = B * T
    x2 = x.reshape(rows, F)
    block_rows = 8192
    out = pl.pallas_call(
        _copy_kernel,
        grid=(rows // block_rows,),
        in_specs=[pl.BlockSpec((block_rows, F), lambda i: (i, 0))],
        out_specs=pl.BlockSpec((block_rows, F), lambda i: (i, 0)),
        out_shape=jax.ShapeDtypeStruct((rows, F), x.dtype),
    )(x2)
    return out.reshape(B, T, F)


# speedup vs baseline: 49.5079x; 1.0061x over previous
import jax
from jax.experimental import pallas as pl


def _copy_kernel(x_ref, o_ref):
    o_ref[...] = x_ref[...]


def kernel(x):
    B, T, F = x.shape
    rows = B * T
    x2 = x.reshape(rows, F)
    block_rows = 8192
    out = pl.pallas_call(
        _copy_kernel,
        grid=(rows // block_rows,),
        in_specs=[pl.BlockSpec((block_rows, F), lambda i: (i, 0))],
        out_specs=pl.BlockSpec((block_rows, F), lambda i: (i, 0)),
        out_shape=jax.ShapeDtypeStruct((rows, F), x.dtype),
    )(x2)
    return out.reshape(B, T, F)
